# baseline (speedup 1.0000x reference)
import jax
import jax.numpy as jnp
from jax import lax
from jax.experimental import pallas as pl
from jax.experimental.pallas import tpu as pltpu
from jax.experimental.pallas import tpu_sc as plsc

NC = 2
NS = 16
NW = NC * NS

VOCAB = 1000000
MAXLEN = 200
EMBED = 64
BATCH = 4096
SEQ = 200
SPLIT = 128
REST = SEQ - SPLIT

K = 2
B_PER_W = BATCH // NW
N_CHUNKS = B_PER_W // K
VREGS = EMBED // 16
NBLK = BATCH // K

_MESH = plsc.VectorSubcoreMesh(core_axis_name="c", subcore_axis_name="s")


def _wid():
    return lax.axis_index("s") * NC + lax.axis_index("c")


def _emb_body(xa_hbm, xb_hbm, tok_hbm, pos_hbm, lin_hbm,
              pos_v, idxa_a, idxb_a, idxa_b, idxb_b,
              tbuf_a, tbuf_b, rows_a, rows_b,
              isem_a, isem_b, gsem_a, gsem_b, outsem):
    base_b = _wid() * B_PER_W
    base_m = _wid() * N_CHUNKS

    pltpu.sync_copy(pos_hbm, pos_v)

    def prefetch(g, idxa, idxb, isem):
        b0 = base_b + g * K
        pltpu.async_copy(xa_hbm.at[pl.ds(b0, K)], idxa, isem)
        pltpu.async_copy(xb_hbm.at[pl.ds(b0, K)], idxb, isem)

    def launch(g, idxa, idxb, tbuf, isem, gsem):
        pltpu.make_async_copy(xa_hbm.at[pl.ds(0, K)], idxa, isem).wait()
        pltpu.make_async_copy(xb_hbm.at[pl.ds(0, K)], idxb, isem).wait()
        for k in range(K):
            pltpu.async_copy(
                tok_hbm.at[idxa.at[k]],
                tbuf.at[pl.ds(k * SEQ, SPLIT)],
                gsem,
            )
            pltpu.async_copy(
                tok_hbm.at[idxb.at[k]],
                tbuf.at[pl.ds(k * SEQ + SPLIT, REST)],
                gsem,
            )

    def drain(tbuf, gsem):
        pltpu.make_async_copy(
            tok_hbm.at[pl.ds(0, K * SEQ)], tbuf, gsem
        ).wait()

    def add(tbuf, rows):
        def body(s):
            for d in range(VREGS):
                pv = pos_v[s, pl.ds(d * 16, 16)]
                for k in range(K):
                    rows[s, pl.ds(k * EMBED + d * 16, 16)] = (
                        tbuf[k * SEQ + s, pl.ds(d * 16, 16)] + pv
                    )
        plsc.parallel_loop(0, SEQ, unroll=2)(body)

    def put(g, rows):
        pltpu.async_copy(rows, lin_hbm.at[base_m + g], outsem)

    def wait_out():
        pltpu.make_async_copy(rows_a, lin_hbm.at[0], outsem).wait()

    prefetch(0, idxa_a, idxb_a, isem_a)
    prefetch(1, idxa_b, idxb_b, isem_b)
    launch(0, idxa_a, idxb_a, tbuf_a, isem_a, gsem_a)
    launch(1, idxa_b, idxb_b, tbuf_b, isem_b, gsem_b)
    drain(tbuf_a, gsem_a)
    prefetch(2, idxa_a, idxb_a, isem_a)
    add(tbuf_a, rows_a)
    put(0, rows_a)

    def body(g2, c):
        g = 1 + 2 * g2
        wait_out()
        launch(g + 1, idxa_a, idxb_a, tbuf_a, isem_a, gsem_a)
        drain(tbuf_b, gsem_b)
        prefetch(g + 2, idxa_b, idxb_b, isem_b)
        add(tbuf_b, rows_b)
        put(g, rows_b)
        wait_out()
        launch(g + 2, idxa_b, idxb_b, tbuf_b, isem_b, gsem_b)
        drain(tbuf_a, gsem_a)
        prefetch(jnp.minimum(g + 3, N_CHUNKS - 1), idxa_a, idxb_a, isem_a)
        add(tbuf_a, rows_a)
        put(g + 1, rows_a)
        return c

    lax.fori_loop(0, (N_CHUNKS - 2) // 2, body, 0)

    drain(tbuf_b, gsem_b)
    add(tbuf_b, rows_b)
    put(N_CHUNKS - 1, rows_b)
    pltpu.make_async_copy(xa_hbm.at[pl.ds(0, K)], idxa_a, isem_a).wait()
    pltpu.make_async_copy(xb_hbm.at[pl.ds(0, K)], idxb_a, isem_a).wait()
    wait_out()
    wait_out()


SS = 2
NIT = SEQ // SS
MB = B_PER_W // K


def _xpose_body(lin_hbm, out_hbm, vbuf_a, vbuf_b, obuf_a, obuf_b,
                rsem_a, rsem_b, osem):
    w = _wid()
    m0 = w * MB
    b0 = w * B_PER_W

    def read(i, vbuf, rsem):
        pltpu.async_copy(
            lin_hbm.at[pl.ds(m0, MB), pl.ds(i * SS, SS), :], vbuf, rsem
        )

    def wait_read(vbuf, rsem):
        pltpu.make_async_copy(
            lin_hbm.at[pl.ds(0, MB), pl.ds(0, SS), :], vbuf, rsem
        ).wait()

    def xpose(vbuf, obuf):
        iota = lax.iota(jnp.int32, 16)

        def jbody(jj, c):
            s1 = jj >> 3
            bvec = ((jj & 7) * 16) + iota
            mvec = bvec >> 1
            colbase = (bvec & 1) * EMBED
            svec = jnp.full((16,), s1, jnp.int32)
            for cc in range(EMBED):
                v = plsc.load_gather(vbuf, [mvec, svec, colbase + cc])
                obuf[s1, cc, pl.ds((jj & 7) * 16, 16)] = v
            return c

        lax.fori_loop(0, SS * 8, jbody, 0)

    def put(i, obuf):
        pltpu.async_copy(
            obuf, out_hbm.at[pl.ds(i * SS, SS), :, pl.ds(b0, 128)], osem
        )

    def wait_out():
        pltpu.make_async_copy(
            obuf_a, out_hbm.at[pl.ds(0, SS), :, pl.ds(0, 128)], osem
        ).wait()

    read(0, vbuf_a, rsem_a)
    read(1, vbuf_b, rsem_b)
    wait_read(vbuf_a, rsem_a)
    xpose(vbuf_a, obuf_a)
    put(0, obuf_a)
    read(2, vbuf_a, rsem_a)
    wait_read(vbuf_b, rsem_b)
    xpose(vbuf_b, obuf_b)
    put(1, obuf_b)
    read(3, vbuf_b, rsem_b)

    def body(t, c):
        i = 2 * t
        wait_out()
        wait_read(vbuf_a, rsem_a)
        xpose(vbuf_a, obuf_a)
        put(i, obuf_a)
        read(jnp.minimum(i + 2, NIT - 1), vbuf_a, rsem_a)
        wait_out()
        wait_read(vbuf_b, rsem_b)
        xpose(vbuf_b, obuf_b)
        put(i + 1, obuf_b)
        read(jnp.minimum(i + 3, NIT - 1), vbuf_b, rsem_b)
        return c

    lax.fori_loop(1, NIT // 2, body, 0)

    wait_out()
    wait_out()
    wait_read(vbuf_a, rsem_a)
    wait_read(vbuf_b, rsem_b)


@jax.jit
def _emb(xa, xb, token_table, pos_table):
    k1 = pl.kernel(
        _emb_body,
        out_type=jax.ShapeDtypeStruct((NBLK, SEQ, K * EMBED), jnp.float32),
        mesh=_MESH,
        scratch_types=[
            pltpu.VMEM((MAXLEN, EMBED), jnp.float32),
            pltpu.VMEM((K, SPLIT), jnp.int32),
            pltpu.VMEM((K, REST), jnp.int32),
            pltpu.VMEM((K, SPLIT), jnp.int32),
            pltpu.VMEM((K, REST), jnp.int32),
            pltpu.VMEM((K * SEQ, EMBED), jnp.float32),
            pltpu.VMEM((K * SEQ, EMBED), jnp.float32),
            pltpu.VMEM((SEQ, K * EMBED), jnp.float32),
            pltpu.VMEM((SEQ, K * EMBED), jnp.float32),
            pltpu.SemaphoreType.DMA,
            pltpu.SemaphoreType.DMA,
            pltpu.SemaphoreType.DMA,
            pltpu.SemaphoreType.DMA,
            pltpu.SemaphoreType.DMA,
        ],
        compiler_params=pltpu.CompilerParams(use_tc_tiling_on_sc=False),
    )
    lin = k1(xa, xb, token_table, pos_table)

    k2 = pl.kernel(
        _xpose_body,
        out_type=jax.ShapeDtypeStruct((SEQ, EMBED, BATCH), jnp.float32),
        mesh=_MESH,
        scratch_types=[
            pltpu.VMEM((MB, SS, K * EMBED), jnp.float32),
            pltpu.VMEM((MB, SS, K * EMBED), jnp.float32),
            pltpu.VMEM((SS, EMBED, 128), jnp.float32),
            pltpu.VMEM((SS, EMBED, 128), jnp.float32),
            pltpu.SemaphoreType.DMA,
            pltpu.SemaphoreType.DMA,
            pltpu.SemaphoreType.DMA,
        ],
        compiler_params=pltpu.CompilerParams(
            use_tc_tiling_on_sc=False, needs_layout_passes=False
        ),
    )
    outT = k2(lin)
    return outT.transpose(2, 0, 1)


def kernel(x, token_table, pos_table):
    x = x.astype(jnp.int32)
    return _emb(x[:, :SPLIT], x[:, SPLIT:], token_table, pos_table)


# speedup vs baseline: 1.7593x; 1.7593x over previous
import jax
import jax.numpy as jnp
from jax import lax
from jax.experimental import pallas as pl
from jax.experimental.pallas import tpu as pltpu
from jax.experimental.pallas import tpu_sc as plsc

NC = 2
NS = 16
NW = NC * NS

VOCAB = 1000000
MAXLEN = 200
EMBED = 64
BATCH = 4096
SEQ = 200
SPLIT = 128
REST = SEQ - SPLIT

K = 2
B_PER_W = BATCH // NW
N_CHUNKS = B_PER_W // K
VREGS = EMBED // 16

_MESH = plsc.VectorSubcoreMesh(core_axis_name="c", subcore_axis_name="s")


def _wid():
    return lax.axis_index("s") * NC + lax.axis_index("c")


def _emb_body(xa_hbm, xb_hbm, tok_hbm, pos_hbm, out_hbm,
              pos_v, idxa_a, idxb_a, idxa_b, idxb_b, rows_a, rows_b,
              isem_a, isem_b, gsem_a, gsem_b, outsem):
    base_b = _wid() * B_PER_W

    pltpu.sync_copy(pos_hbm, pos_v)

    def prefetch(g, idxa, idxb, isem):
        b0 = base_b + g * K
        pltpu.async_copy(xa_hbm.at[pl.ds(b0, K)], idxa, isem)
        pltpu.async_copy(xb_hbm.at[pl.ds(b0, K)], idxb, isem)

    def launch(g, idxa, idxb, rows, isem, gsem):
        pltpu.make_async_copy(xa_hbm.at[pl.ds(0, K)], idxa, isem).wait()
        pltpu.make_async_copy(xb_hbm.at[pl.ds(0, K)], idxb, isem).wait()
        for k in range(K):
            pltpu.async_copy(
                tok_hbm.at[idxa.at[k]], rows.at[k, pl.ds(0, SPLIT)], gsem
            )
            pltpu.async_copy(
                tok_hbm.at[idxb.at[k]], rows.at[k, pl.ds(SPLIT, REST)], gsem
            )

    def drain(rows, gsem):
        pltpu.make_async_copy(out_hbm.at[pl.ds(0, K)], rows, gsem).wait()

    def add(rows):
        def body(s):
            for d in range(VREGS):
                pv = pos_v[s, pl.ds(d * 16, 16)]
                for k in range(K):
                    rows[k, s, pl.ds(d * 16, 16)] = (
                        rows[k, s, pl.ds(d * 16, 16)] + pv
                    )
        plsc.parallel_loop(0, SEQ, unroll=2)(body)

    def put(g, rows):
        b0 = base_b + g * K
        pltpu.async_copy(rows, out_hbm.at[pl.ds(b0, K)], outsem)

    def wait_out():
        pltpu.make_async_copy(rows_a, out_hbm.at[pl.ds(0, K)], outsem).wait()

    prefetch(0, idxa_a, idxb_a, isem_a)
    prefetch(1, idxa_b, idxb_b, isem_b)
    launch(0, idxa_a, idxb_a, rows_a, isem_a, gsem_a)
    launch(1, idxa_b, idxb_b, rows_b, isem_b, gsem_b)
    drain(rows_a, gsem_a)
    prefetch(2, idxa_a, idxb_a, isem_a)
    add(rows_a)
    put(0, rows_a)

    def body(g2, c):
        g = 1 + 2 * g2
        wait_out()
        launch(g + 1, idxa_a, idxb_a, rows_a, isem_a, gsem_a)
        drain(rows_b, gsem_b)
        prefetch(g + 2, idxa_b, idxb_b, isem_b)
        add(rows_b)
        put(g, rows_b)
        wait_out()
        launch(g + 2, idxa_b, idxb_b, rows_b, isem_b, gsem_b)
        drain(rows_a, gsem_a)
        prefetch(jnp.minimum(g + 3, N_CHUNKS - 1), idxa_a, idxb_a, isem_a)
        add(rows_a)
        put(g + 1, rows_a)
        return c

    lax.fori_loop(0, (N_CHUNKS - 2) // 2, body, 0)

    drain(rows_b, gsem_b)
    add(rows_b)
    put(N_CHUNKS - 1, rows_b)
    pltpu.make_async_copy(xa_hbm.at[pl.ds(0, K)], idxa_a, isem_a).wait()
    pltpu.make_async_copy(xb_hbm.at[pl.ds(0, K)], idxb_a, isem_a).wait()
    wait_out()
    wait_out()


@jax.jit
def _emb(xa, xb, token_table, pos_table):
    k1 = pl.kernel(
        _emb_body,
        out_type=jax.ShapeDtypeStruct((BATCH, SEQ, EMBED), jnp.float32),
        mesh=_MESH,
        scratch_types=[
            pltpu.VMEM((MAXLEN, EMBED), jnp.float32),
            pltpu.VMEM((K, SPLIT), jnp.int32),
            pltpu.VMEM((K, REST), jnp.int32),
            pltpu.VMEM((K, SPLIT), jnp.int32),
            pltpu.VMEM((K, REST), jnp.int32),
            pltpu.VMEM((K, SEQ, EMBED), jnp.float32),
            pltpu.VMEM((K, SEQ, EMBED), jnp.float32),
            pltpu.SemaphoreType.DMA,
            pltpu.SemaphoreType.DMA,
            pltpu.SemaphoreType.DMA,
            pltpu.SemaphoreType.DMA,
            pltpu.SemaphoreType.DMA,
        ],
        compiler_params=pltpu.CompilerParams(use_tc_tiling_on_sc=False),
    )
    return k1(xa, xb, token_table, pos_table)


def kernel(x, token_table, pos_table):
    x = x.astype(jnp.int32)
    return _emb(x[:, :SPLIT], x[:, SPLIT:], token_table, pos_table)


# speedup vs baseline: 1.7727x; 1.0076x over previous
import jax
import jax.numpy as jnp
from jax import lax
from jax.experimental import pallas as pl
from jax.experimental.pallas import tpu as pltpu
from jax.experimental.pallas import tpu_sc as plsc

NC = 2
NS = 16
NW = NC * NS

VOCAB = 1000000
MAXLEN = 200
EMBED = 64
BATCH = 4096
SEQ = 200
SPLIT = 128
REST = SEQ - SPLIT

K = 4
B_PER_W = BATCH // NW
N_CHUNKS = B_PER_W // K
VREGS = EMBED // 16

_MESH = plsc.VectorSubcoreMesh(core_axis_name="c", subcore_axis_name="s")


def _wid():
    return lax.axis_index("s") * NC + lax.axis_index("c")


def _emb_body(xa_hbm, xb_hbm, tok_hbm, pos_hbm, out_hbm,
              pos_v, idxa_a, idxb_a, idxa_b, idxb_b, rows_a, rows_b,
              isem_a, isem_b, gsem_a, gsem_b, outsem):
    base_b = _wid() * B_PER_W

    pltpu.sync_copy(pos_hbm, pos_v)

    def prefetch(g, idxa, idxb, isem):
        b0 = base_b + g * K
        pltpu.async_copy(xa_hbm.at[pl.ds(b0, K)], idxa, isem)
        pltpu.async_copy(xb_hbm.at[pl.ds(b0, K)], idxb, isem)

    def launch(g, idxa, idxb, rows, isem, gsem):
        pltpu.make_async_copy(xa_hbm.at[pl.ds(0, K)], idxa, isem).wait()
        pltpu.make_async_copy(xb_hbm.at[pl.ds(0, K)], idxb, isem).wait()
        for k in range(K):
            pltpu.async_copy(
                tok_hbm.at[idxa.at[k]], rows.at[k, pl.ds(0, SPLIT)], gsem
            )
            pltpu.async_copy(
                tok_hbm.at[idxb.at[k]], rows.at[k, pl.ds(SPLIT, REST)], gsem
            )

    def drain(rows, gsem):
        pltpu.make_async_copy(out_hbm.at[pl.ds(0, K)], rows, gsem).wait()

    def add(rows):
        def body(s):
            for d in range(VREGS):
                pv = pos_v[s, pl.ds(d * 16, 16)]
                for k in range(K):
                    rows[k, s, pl.ds(d * 16, 16)] = (
                        rows[k, s, pl.ds(d * 16, 16)] + pv
                    )
        plsc.parallel_loop(0, SEQ, unroll=2)(body)

    def put(g, rows):
        b0 = base_b + g * K
        pltpu.async_copy(rows, out_hbm.at[pl.ds(b0, K)], outsem)

    def wait_out():
        pltpu.make_async_copy(rows_a, out_hbm.at[pl.ds(0, K)], outsem).wait()

    prefetch(0, idxa_a, idxb_a, isem_a)
    prefetch(1, idxa_b, idxb_b, isem_b)
    launch(0, idxa_a, idxb_a, rows_a, isem_a, gsem_a)
    launch(1, idxa_b, idxb_b, rows_b, isem_b, gsem_b)
    drain(rows_a, gsem_a)
    prefetch(2, idxa_a, idxb_a, isem_a)
    add(rows_a)
    put(0, rows_a)

    def body(g2, c):
        g = 1 + 2 * g2
        wait_out()
        launch(g + 1, idxa_a, idxb_a, rows_a, isem_a, gsem_a)
        drain(rows_b, gsem_b)
        prefetch(g + 2, idxa_b, idxb_b, isem_b)
        add(rows_b)
        put(g, rows_b)
        wait_out()
        launch(g + 2, idxa_b, idxb_b, rows_b, isem_b, gsem_b)
        drain(rows_a, gsem_a)
        prefetch(jnp.minimum(g + 3, N_CHUNKS - 1), idxa_a, idxb_a, isem_a)
        add(rows_a)
        put(g + 1, rows_a)
        return c

    lax.fori_loop(0, (N_CHUNKS - 2) // 2, body, 0)

    drain(rows_b, gsem_b)
    add(rows_b)
    put(N_CHUNKS - 1, rows_b)
    pltpu.make_async_copy(xa_hbm.at[pl.ds(0, K)], idxa_a, isem_a).wait()
    pltpu.make_async_copy(xb_hbm.at[pl.ds(0, K)], idxb_a, isem_a).wait()
    wait_out()
    wait_out()


@jax.jit
def _emb(xa, xb, token_table, pos_table):
    k1 = pl.kernel(
        _emb_body,
        out_type=jax.ShapeDtypeStruct((BATCH, SEQ, EMBED), jnp.float32),
        mesh=_MESH,
        scratch_types=[
            pltpu.VMEM((MAXLEN, EMBED), jnp.float32),
            pltpu.VMEM((K, SPLIT), jnp.int32),
            pltpu.VMEM((K, REST), jnp.int32),
            pltpu.VMEM((K, SPLIT), jnp.int32),
            pltpu.VMEM((K, REST), jnp.int32),
            pltpu.VMEM((K, SEQ, EMBED), jnp.float32),
            pltpu.VMEM((K, SEQ, EMBED), jnp.float32),
            pltpu.SemaphoreType.DMA,
            pltpu.SemaphoreType.DMA,
            pltpu.SemaphoreType.DMA,
            pltpu.SemaphoreType.DMA,
            pltpu.SemaphoreType.DMA,
        ],
        compiler_params=pltpu.CompilerParams(use_tc_tiling_on_sc=False),
    )
    return k1(xa, xb, token_table, pos_table)


def kernel(x, token_table, pos_table):
    x = x.astype(jnp.int32)
    return _emb(x[:, :SPLIT], x[:, SPLIT:], token_table, pos_table)
